# baseline (speedup 1.0000x reference)
import functools

import jax
import jax.numpy as jnp
from jax import lax
from jax.experimental import pallas as pl
from jax.experimental.pallas import tpu as pltpu
from jax.experimental.pallas import tpu_sc as plsc


def _make_sc_gather_t(D, V, B):
    info = plsc.get_sparse_core_info()
    NC, NS, L = info.num_cores, info.num_subcores, info.num_lanes
    NW = NC * NS
    assert D % NW == 0
    rows_per_w = D // NW
    B_CH = 4096 if B % 4096 == 0 else B
    n_ch = B // B_CH
    mesh = plsc.VectorSubcoreMesh(core_axis_name="c", subcore_axis_name="s")

    @functools.partial(
        pl.kernel,
        mesh=mesh,
        out_type=jax.ShapeDtypeStruct((D, B), jnp.float32),
        scratch_types=[
            pltpu.VMEM((V,), jnp.float32),
            pltpu.VMEM((B,), jnp.int32),
            pltpu.VMEM((B_CH,), jnp.float32),
            pltpu.VMEM((B_CH,), jnp.float32),
            pltpu.SemaphoreType.DMA,
            pltpu.SemaphoreType.DMA,
            pltpu.SemaphoreType.DMA,
            pltpu.SemaphoreType.DMA,
        ],
        compiler_params=pltpu.CompilerParams(needs_layout_passes=False),
    )
    def gather_k(tbl_hbm, idx_hbm, out_hbm, row_v, idx_v, out_v0, out_v1,
                 sem_i, sem_r, sem_o0, sem_o1):
        wid = lax.axis_index("s") * NC + lax.axis_index("c")
        pltpu.sync_copy(idx_hbm, idx_v)

        def row_body(rr, carry):
            d = wid * rows_per_w + rr
            pltpu.sync_copy(tbl_hbm.at[d], row_v)

            def ch_body(c, carry2):
                base = c * B_CH

                @plsc.parallel_loop(0, B_CH, L, unroll=8)
                def body(j):
                    idxs = idx_v[pl.ds(base + j, L)]
                    out_v0[pl.ds(j, L)] = plsc.load_gather(row_v, [idxs])

                pltpu.sync_copy(out_v0, out_hbm.at[d, pl.ds(base, B_CH)])
                return carry2

            lax.fori_loop(0, n_ch, ch_body, 0)
            return carry

        lax.fori_loop(0, rows_per_w, row_body, 0)

    return gather_k


def _proj_t_body(w_ref, e_ref, b_ref, o_ref):
    o_ref[...] = (
        lax.dot_general(
            w_ref[...],
            e_ref[...],
            (((0,), (0,)), ((), ())),
            preferred_element_type=jnp.float32,
        )
        + b_ref[...]
    )


def _make_tc_proj_t(D, B, blk):
    return pl.pallas_call(
        _proj_t_body,
        grid=(B // blk,),
        in_specs=[
            pl.BlockSpec((D, D), lambda i: (0, 0)),
            pl.BlockSpec((D, blk), lambda i: (0, i)),
            pl.BlockSpec((D, 1), lambda i: (0, 0)),
        ],
        out_specs=pl.BlockSpec((D, blk), lambda i: (0, i)),
        out_shape=jax.ShapeDtypeStruct((D, B), jnp.float32),
    )


def kernel(y, embed_table, W, b):
    V, D = embed_table.shape
    B = y.shape[0]
    tbl_t = embed_table.T
    e_t = _make_sc_gather_t(D, V, B)(tbl_t, y.astype(jnp.int32))
    out_t = _make_tc_proj_t(D, B, blk=B)(W, e_t, b.reshape(D, 1))
    return out_t.T


# speedup vs baseline: 1.0309x; 1.0309x over previous
import functools

import jax
import jax.numpy as jnp
from jax import lax
from jax.experimental import pallas as pl
from jax.experimental.pallas import tpu as pltpu
from jax.experimental.pallas import tpu_sc as plsc


def _make_sc_gather_t(D, V, B):
    info = plsc.get_sparse_core_info()
    NC, NS, L = info.num_cores, info.num_subcores, info.num_lanes
    NW = NC * NS
    assert D % NW == 0
    rows_per_w = D // NW
    B_CH = 8192 if B % 8192 == 0 else B
    n_ch = B // B_CH
    mesh = plsc.VectorSubcoreMesh(core_axis_name="c", subcore_axis_name="s")

    @functools.partial(
        pl.kernel,
        mesh=mesh,
        out_type=jax.ShapeDtypeStruct((D, B), jnp.float32),
        scratch_types=[
            pltpu.VMEM((V,), jnp.float32),
            pltpu.VMEM((B,), jnp.int32),
            pltpu.VMEM((B_CH,), jnp.float32),
        ],
        compiler_params=pltpu.CompilerParams(needs_layout_passes=False),
    )
    def gather_k(tbl_hbm, idx_hbm, out_hbm, row_v, idx_v, out_v0):
        wid = lax.axis_index("s") * NC + lax.axis_index("c")
        pltpu.sync_copy(idx_hbm, idx_v)

        def row_body(rr, carry):
            d = wid * rows_per_w + rr
            pltpu.sync_copy(tbl_hbm.at[d], row_v)

            def ch_body(c, carry2):
                base = c * B_CH

                @plsc.parallel_loop(0, B_CH, L, unroll=8)
                def body(j):
                    idxs = idx_v[pl.ds(base + j, L)]
                    out_v0[pl.ds(j, L)] = plsc.load_gather(row_v, [idxs])

                pltpu.sync_copy(out_v0, out_hbm.at[d, pl.ds(base, B_CH)])
                return carry2

            lax.fori_loop(0, n_ch, ch_body, 0)
            return carry

        lax.fori_loop(0, rows_per_w, row_body, 0)

    return gather_k


def _proj_t_body(w_ref, e_ref, b_ref, o_ref):
    o_ref[...] = (
        lax.dot_general(
            w_ref[...],
            e_ref[...],
            (((0,), (0,)), ((), ())),
            preferred_element_type=jnp.float32,
        )
        + b_ref[...]
    )


def _make_tc_proj_t(D, B, blk):
    return pl.pallas_call(
        _proj_t_body,
        grid=(B // blk,),
        in_specs=[
            pl.BlockSpec((D, D), lambda i: (0, 0)),
            pl.BlockSpec((D, blk), lambda i: (0, i)),
            pl.BlockSpec((D, 1), lambda i: (0, 0)),
        ],
        out_specs=pl.BlockSpec((D, blk), lambda i: (0, i)),
        out_shape=jax.ShapeDtypeStruct((D, B), jnp.float32),
    )


def kernel(y, embed_table, W, b):
    V, D = embed_table.shape
    B = y.shape[0]
    tbl_t = embed_table.T
    e_t = _make_sc_gather_t(D, V, B)(tbl_t, y.astype(jnp.int32))
    out_t = _make_tc_proj_t(D, B, blk=8192)(W, e_t, b.reshape(D, 1))
    return out_t.T
